# Initial kernel scaffold; baseline (speedup 1.0000x reference)
import jax
import jax.numpy as jnp
from jax.experimental import pallas as pl


def kernel(activations):
    raise NotImplementedError("write your pallas kernel here")


# speedup vs baseline: 20.9528x; 20.9528x over previous
import functools

import jax
import jax.numpy as jnp
from jax import lax
from jax.experimental import pallas as pl
from jax.experimental.pallas import tpu as pltpu
from jax.experimental.pallas import tpu_sc as plsc

B = 128
N = 32768
K = 8
LANES = 16
NUM_CORES = 2
NUM_SUBCORES = 16
NUM_WORKERS = NUM_CORES * NUM_SUBCORES
COLS_PER_WORKER = N // NUM_WORKERS
W = 512
CHUNKS = COLS_PER_WORKER // W
GROUPS = W // LANES
ROW_BLOCKS = B // K

_SORT8 = ((0, 1), (2, 3), (4, 5), (6, 7),
          (0, 2), (1, 3), (4, 6), (5, 7),
          (1, 2), (5, 6),
          (0, 4), (1, 5), (2, 6), (3, 7),
          (2, 4), (3, 5),
          (1, 2), (3, 4), (5, 6))
_BITONIC8 = ((0, 4), (1, 5), (2, 6), (3, 7),
             (0, 2), (1, 3), (4, 6), (5, 7),
             (0, 1), (2, 3), (4, 5), (6, 7))


def _net_desc(vals, net):
    vals = list(vals)
    for i, j in net:
        hi = jnp.maximum(vals[i], vals[j])
        lo = jnp.minimum(vals[i], vals[j])
        vals[i], vals[j] = hi, lo
    return vals


def _topk_mask_body(a_hbm, out_hbm, buf):
    wid = lax.axis_index("s") * NUM_CORES + lax.axis_index("c")
    base = wid * COLS_PER_WORKER

    for chunk in range(CHUNKS):
        c0 = base + chunk * W
        pltpu.sync_copy(a_hbm.at[:, pl.ds(c0, W)], buf)

        def group_body(g, _):
            col = g * LANES

            def load8(rb):
                return [buf[rb * K + j, pl.ds(col, LANES)] for j in range(K)]

            run = _net_desc(load8(0), _SORT8)

            def blk_body(rb, run):
                s = _net_desc(
                    [buf[rb * K + j, pl.ds(col, LANES)] for j in range(K)],
                    _SORT8)
                merged = [jnp.maximum(run[i], s[K - 1 - i]) for i in range(K)]
                return tuple(_net_desc(merged, _BITONIC8))

            run = lax.fori_loop(1, ROW_BLOCKS, blk_body, tuple(run))
            thr = run[K - 1]
            zero = jnp.zeros((LANES,), jnp.float32)

            def mask_body(rb, _):
                for j in range(K):
                    r = rb * K + j
                    v = buf[r, pl.ds(col, LANES)]
                    buf[r, pl.ds(col, LANES)] = jnp.where(v >= thr, v, zero)
                return 0

            lax.fori_loop(0, ROW_BLOCKS, mask_body, 0)
            return 0

        lax.fori_loop(0, GROUPS, group_body, 0)
        pltpu.sync_copy(buf, out_hbm.at[:, pl.ds(c0, W)])


@jax.jit
def _topk_mask(activations):
    mesh = plsc.VectorSubcoreMesh(core_axis_name="c", subcore_axis_name="s")
    f = functools.partial(
        pl.kernel,
        out_type=jax.ShapeDtypeStruct((B, N), jnp.float32),
        mesh=mesh,
        scratch_types=[pltpu.VMEM((B, W), jnp.float32)],
    )(_topk_mask_body)
    return f(activations)


def kernel(activations):
    return _topk_mask(activations)[:, :, None, None]
